# baseline (speedup 1.0000x reference)
import functools

import jax
import jax.numpy as jnp
from jax import lax
from jax.experimental import pallas as pl
from jax.experimental.pallas import tpu as pltpu
from jax.experimental.pallas import tpu_sc as plsc

N = 10000
E = 320000
H = 256
OUT = 1
BE = 2560

NS = 16
NFT = 10
RPT = N // NFT
BN = 2000

KS = 80
SCH = (E // NS) // KS

_sc_mesh = plsc.VectorSubcoreMesh(core_axis_name="c", subcore_axis_name="s")


def _edge_mm0_kernel(ea_ref, wm_ref, we_ref, bm_ref, be_ref, em_ref, ee_ref):
    ea = ea_ref[...]
    em_ref[...] = jnp.dot(ea, wm_ref[...],
                          preferred_element_type=jnp.float32) + bm_ref[...]
    ee_ref[...] = (jnp.dot(ea, we_ref[...], preferred_element_type=jnp.float32)
                   + be_ref[...]).astype(jnp.bfloat16)


def _edge_mm0(ea, wcm, wce, bcm, bce):
    de = ea.shape[1]
    return pl.pallas_call(
        _edge_mm0_kernel,
        grid=(E // BE,),
        in_specs=[
            pl.BlockSpec((BE, de), lambda i: (i, 0)),
            pl.BlockSpec((de, H), lambda i: (0, 0)),
            pl.BlockSpec((de, H), lambda i: (0, 0)),
            pl.BlockSpec((1, H), lambda i: (0, 0)),
            pl.BlockSpec((1, H), lambda i: (0, 0)),
        ],
        out_specs=[
            pl.BlockSpec((BE, H), lambda i: (i, 0)),
            pl.BlockSpec((BE, H), lambda i: (i, 0)),
        ],
        out_shape=[
            jax.ShapeDtypeStruct((E, H), jnp.float32),
            jax.ShapeDtypeStruct((E, H), jnp.bfloat16),
        ],
    )(ea, wcm, wce, bcm, bce)


def _edge_mm_kernel(ee_ref, gs_ref, gd_ref, be_ref, wm_ref, we_ref,
                    em_ref, eeo_ref):
    e = jax.nn.relu(
        ee_ref[...].astype(jnp.float32)
        + gs_ref[...].astype(jnp.float32)
        + gd_ref[...].astype(jnp.float32)
        + be_ref[...])
    em_ref[...] = jnp.dot(e, wm_ref[...], preferred_element_type=jnp.float32)
    if eeo_ref is not None:
        eeo_ref[...] = jnp.dot(
            e, we_ref[...], preferred_element_type=jnp.float32
        ).astype(jnp.bfloat16)


def _edge_mm(ee_prev, gs, gd, b_e, w_msg, w_ee, want_ee):
    nblk = E // BE
    out_specs = [pl.BlockSpec((BE, H), lambda i: (i, 0))]
    out_shape = [jax.ShapeDtypeStruct((E, H), jnp.float32)]
    in_specs = [
        pl.BlockSpec((BE, H), lambda i: (i, 0)),
        pl.BlockSpec((BE, H), lambda i: (i, 0)),
        pl.BlockSpec((BE, H), lambda i: (i, 0)),
        pl.BlockSpec((1, H), lambda i: (0, 0)),
        pl.BlockSpec((H, H), lambda i: (0, 0)),
    ]
    args = [ee_prev, gs, gd, b_e, w_msg]
    if want_ee:
        out_specs.append(pl.BlockSpec((BE, H), lambda i: (i, 0)))
        out_shape.append(jax.ShapeDtypeStruct((E, H), jnp.bfloat16))
        in_specs.append(pl.BlockSpec((H, H), lambda i: (0, 0)))
        args.append(w_ee)
        body = _edge_mm_kernel
    else:
        def body(ee_ref, gs_ref, gd_ref, be_ref, wm_ref, em_ref):
            _edge_mm_kernel(ee_ref, gs_ref, gd_ref, be_ref, wm_ref, None,
                            em_ref, None)
    res = pl.pallas_call(
        body,
        grid=(nblk,),
        in_specs=in_specs,
        out_specs=out_specs,
        out_shape=out_shape,
    )(*args)
    return res if want_ee else (res[0], None)


def _prep_kernel(x_ref, wn_ref, bn_ref, wm_ref, h_ref, hm_ref):
    h = jnp.dot(x_ref[...], wn_ref[...],
                preferred_element_type=jnp.float32) + bn_ref[...]
    h_ref[...] = h
    hm_ref[...] = jnp.dot(h, wm_ref[...], preferred_element_type=jnp.float32)


def _prep(x, w_ne, b_ne, w_msg):
    df = x.shape[1]
    return pl.pallas_call(
        _prep_kernel,
        grid=(N // BN,),
        in_specs=[
            pl.BlockSpec((BN, df), lambda i: (i, 0)),
            pl.BlockSpec((df, H), lambda i: (0, 0)),
            pl.BlockSpec((1, H), lambda i: (0, 0)),
            pl.BlockSpec((H, H), lambda i: (0, 0)),
        ],
        out_specs=[
            pl.BlockSpec((BN, H), lambda i: (i, 0)),
            pl.BlockSpec((BN, H), lambda i: (i, 0)),
        ],
        out_shape=[
            jax.ShapeDtypeStruct((N, H), jnp.float32),
            jax.ShapeDtypeStruct((N, H), jnp.float32),
        ],
    )(x, w_ne, b_ne, w_msg)


def _node_mid_kernel(h_ref, aga_ref, agb_ref, ws_ref, bh_ref, wm_ref,
                     wes_ref, wed_ref, hn_ref, hm_ref, hs_ref, hd_ref):
    hn = jax.nn.relu(
        jnp.dot(h_ref[...], ws_ref[...], preferred_element_type=jnp.float32)
        + jnp.concatenate([aga_ref[...], agb_ref[...]], axis=1)
        + bh_ref[...])
    hn_ref[...] = hn
    if hm_ref is not None:
        hm_ref[...] = jnp.dot(hn, wm_ref[...],
                              preferred_element_type=jnp.float32)
    hs_ref[...] = jnp.dot(
        hn, wes_ref[...], preferred_element_type=jnp.float32
    ).astype(jnp.bfloat16)
    hd_ref[...] = jnp.dot(
        hn, wed_ref[...], preferred_element_type=jnp.float32
    ).astype(jnp.bfloat16)


def _node_mid(h, agg_flat, w_self, b_h, w_msg_next, we_s, we_d):
    nblk = N // BN
    return pl.pallas_call(
        _node_mid_kernel,
        grid=(nblk,),
        in_specs=[
            pl.BlockSpec((BN, H), lambda i: (i, 0)),
            pl.BlockSpec((BN, 128), lambda i: (i, 0)),
            pl.BlockSpec((BN, 128), lambda i: (i + nblk, 0)),
            pl.BlockSpec((H, H), lambda i: (0, 0)),
            pl.BlockSpec((1, H), lambda i: (0, 0)),
            pl.BlockSpec((H, H), lambda i: (0, 0)),
            pl.BlockSpec((H, H), lambda i: (0, 0)),
            pl.BlockSpec((H, H), lambda i: (0, 0)),
        ],
        out_specs=[pl.BlockSpec((BN, H), lambda i: (i, 0))] * 4,
        out_shape=[jax.ShapeDtypeStruct((N, H), jnp.float32)] * 2
        + [jax.ShapeDtypeStruct((N, H), jnp.bfloat16)] * 2,
    )(h, agg_flat, agg_flat, w_self, b_h, w_msg_next, we_s, we_d)


def _node_last_kernel(h_ref, aga_ref, agb_ref, ws_ref, bh_ref, wp_ref,
                      o_ref):
    hn = jax.nn.relu(
        jnp.dot(h_ref[...], ws_ref[...], preferred_element_type=jnp.float32)
        + jnp.concatenate([aga_ref[...], agb_ref[...]], axis=1)
        + bh_ref[...])
    o_ref[...] = jnp.dot(hn, wp_ref[...], preferred_element_type=jnp.float32)


def _node_last(h, agg_flat, w_self, b_h, w_pred_pad):
    nblk = N // BN
    return pl.pallas_call(
        _node_last_kernel,
        grid=(nblk,),
        in_specs=[
            pl.BlockSpec((BN, H), lambda i: (i, 0)),
            pl.BlockSpec((BN, 128), lambda i: (i, 0)),
            pl.BlockSpec((BN, 128), lambda i: (i + nblk, 0)),
            pl.BlockSpec((H, H), lambda i: (0, 0)),
            pl.BlockSpec((1, H), lambda i: (0, 0)),
            pl.BlockSpec((H, 128), lambda i: (0, 0)),
        ],
        out_specs=[pl.BlockSpec((BN, 128), lambda i: (i, 0))],
        out_shape=[jax.ShapeDtypeStruct((N, 128), jnp.float32)],
    )(h, agg_flat, agg_flat, w_self, b_h, w_pred_pad)[0]


def _off(v, m):
    return pl.multiple_of(v, m)


@functools.partial(
    pl.kernel,
    out_type=jax.ShapeDtypeStruct((2 * N, 128), jnp.float32),
    mesh=_sc_mesh,
    scratch_types=[
        pltpu.VMEM((KS,), jnp.int32),
        pltpu.VMEM((KS,), jnp.int32),
        pltpu.VMEM((KS,), jnp.int32),
        pltpu.VMEM((KS,), jnp.int32),
        pltpu.VMEM((KS, 128), jnp.float32),
        pltpu.VMEM((KS, 128), jnp.float32),
        pltpu.VMEM((KS, 128), jnp.float32),
        pltpu.VMEM((KS, 128), jnp.float32),
        pltpu.VMEM_SHARED((N, 128), jnp.float32),
    ] + [pltpu.SemaphoreType.DMA] * 8,
)
def _sc_scatter(hm_hbm, em_hbm, src2_hbm, dst_hbm, z_hbm, out_hbm,
                si0, si1, di0, di1, ga0, ga1, em0, em1, acc,
                ssi0, ssi1, sdi0, sdi1, sg0, sg1, se0, se1):
    c = lax.axis_index("c")
    s = lax.axis_index("s")

    @pl.when(s < NFT)
    def _():
        pltpu.sync_copy(z_hbm, acc.at[pl.ds(s * RPT, RPT)])

    tbase = (c * NS + s) * SCH
    dbase = s * SCH

    def sioff(j):
        return _off(jnp.minimum(tbase + j, tbase + SCH - 1) * KS, 8)

    def dioff(j):
        return _off(jnp.minimum(dbase + j, dbase + SCH - 1) * KS, 8)

    def si_start(j, si, sem):
        pltpu.async_copy(src2_hbm.at[pl.ds(sioff(j), KS)], si, sem)

    def si_wait(j, si, sem):
        pltpu.make_async_copy(src2_hbm.at[pl.ds(sioff(j), KS)], si, sem).wait()

    def di_start(j, di, sem):
        pltpu.async_copy(dst_hbm.at[pl.ds(dioff(j), KS)], di, sem)

    def di_wait(j, di, sem):
        pltpu.make_async_copy(dst_hbm.at[pl.ds(dioff(j), KS)], di, sem).wait()

    def g_start(j, si, ga, em, sg, se):
        pltpu.async_copy(hm_hbm.at[si], ga, sg)
        pltpu.async_copy(
            em_hbm.at[pl.ds(dioff(j), KS), pl.ds(_off(c * 128, 128), 128)],
            em, se)

    def g_wait(j, si, ga, em, sg, se):
        pltpu.make_async_copy(hm_hbm.at[si], ga, sg).wait()
        pltpu.make_async_copy(
            em_hbm.at[pl.ds(dioff(j), KS), pl.ds(_off(c * 128, 128), 128)],
            em, se).wait()

    si_start(0, si0, ssi0)
    si_start(1, si1, ssi1)
    di_start(0, di0, sdi0)
    di_start(1, di1, sdi1)
    si_wait(0, si0, ssi0)
    g_start(0, si0, ga0, em0, sg0, se0)
    si_wait(1, si1, ssi1)
    g_start(1, si1, ga1, em1, sg1, se1)
    plsc.subcore_barrier()

    def chunk(j, si, di, ga, em, ssi, sdi, sg, se, jn):
        g_wait(j, si, ga, em, sg, se)
        si_start(jn, si, ssi)
        di_wait(j, di, sdi)
        pltpu.sync_copy(ga, acc.at[di], add=True)
        pltpu.sync_copy(em, acc.at[di], add=True)
        di_start(jn, di, sdi)
        si_wait(jn, si, ssi)
        g_start(jn, si, ga, em, sg, se)

    @pl.loop(0, SCH, step=2)
    def _(j):
        chunk(j, si0, di0, ga0, em0, ssi0, sdi0, sg0, se0, j + 2)
        chunk(j + 1, si1, di1, ga1, em1, ssi1, sdi1, sg1, se1, j + 3)

    g_wait(SCH, si0, ga0, em0, sg0, se0)
    g_wait(SCH + 1, si1, ga1, em1, sg1, se1)
    di_wait(SCH, di0, sdi0)
    di_wait(SCH + 1, di1, sdi1)

    plsc.subcore_barrier()

    @pl.when(s < NFT)
    def _():
        pltpu.sync_copy(acc.at[pl.ds(s * RPT, RPT)],
                        out_hbm.at[pl.ds(c * N + s * RPT, RPT)])


KG = 40
SUP = 5
SROWS = SUP * KG
EPTG = E // (2 * NS)
SEC = EPTG // SROWS


@functools.partial(
    pl.kernel,
    out_type=[jax.ShapeDtypeStruct((E, 128), jnp.int32),
              jax.ShapeDtypeStruct((E, 128), jnp.int32)],
    mesh=_sc_mesh,
    scratch_types=[
        pltpu.VMEM((SROWS,), jnp.int32),
        pltpu.VMEM((SROWS,), jnp.int32),
        pltpu.VMEM((SROWS,), jnp.int32),
        pltpu.VMEM((SROWS,), jnp.int32),
        pltpu.VMEM((2 * SROWS, 128), jnp.int32),
        pltpu.VMEM((2 * SROWS, 128), jnp.int32),
    ] + [pltpu.SemaphoreType.DMA] * 8,
)
def _sc_gather(hs_hbm, hd_hbm, src_hbm, dst_hbm, gs_hbm, gd_hbm,
               sia0, sia1, sib0, sib1, ra, rb,
               sx0, sx1, sga0, sga1, sgb0, sgb1, sf0, sf1):
    c = lax.axis_index("c")
    s = lax.axis_index("s")
    ebase = (c * NS + s) * EPTG

    def eoff(S):
        return _off(ebase + jnp.minimum(S, SEC - 1) * SROWS, 8)

    def i_start(S, sia, sib, sx):
        pltpu.async_copy(src_hbm.at[pl.ds(eoff(S), SROWS)], sia, sx)
        pltpu.async_copy(dst_hbm.at[pl.ds(eoff(S), SROWS)], sib, sx)

    def i_wait(S, sia, sib, sx):
        pltpu.make_async_copy(
            src_hbm.at[pl.ds(eoff(S), SROWS)], sia, sx).wait()
        pltpu.make_async_copy(
            dst_hbm.at[pl.ds(eoff(S), SROWS)], sib, sx).wait()

    def g_descs(P, sia, sib, sga, sgb):
        for kk in range(SUP):
            yield (hs_hbm.at[sia.at[pl.ds(kk * KG, KG)]],
                   ra.at[pl.ds(P * SROWS + kk * KG, KG)], sga)
            yield (hd_hbm.at[sib.at[pl.ds(kk * KG, KG)]],
                   rb.at[pl.ds(P * SROWS + kk * KG, KG)], sgb)

    def f_descs(S, P, sf):
        yield (ra.at[pl.ds(P * SROWS, SROWS)],
               gs_hbm.at[pl.ds(eoff(S), SROWS)], sf)
        yield (rb.at[pl.ds(P * SROWS, SROWS)],
               gd_hbm.at[pl.ds(eoff(S), SROWS)], sf)

    def section(S, P, sia, sib, sx, sia_n, sib_n, sx_n, sga, sgb, sf,
                wait_f):
        if wait_f:
            for src_r, dst_r, sem in f_descs(S - 2, P, sf):
                pltpu.make_async_copy(src_r, dst_r, sem).wait()
        i_wait(S, sia, sib, sx)
        for src_r, dst_r, sem in g_descs(P, sia, sib, sga, sgb):
            pltpu.async_copy(src_r, dst_r, sem)
        i_start(S + 1, sia_n, sib_n, sx_n)
        for src_r, dst_r, sem in g_descs(P, sia, sib, sga, sgb):
            pltpu.make_async_copy(src_r, dst_r, sem).wait()
        for src_r, dst_r, sem in f_descs(S, P, sf):
            pltpu.async_copy(src_r, dst_r, sem)

    def sec0(S, wait_f):
        section(S, 0, sia0, sib0, sx0, sia1, sib1, sx1, sga0, sgb0, sf0,
                wait_f)

    def sec1(S, wait_f):
        section(S, 1, sia1, sib1, sx1, sia0, sib0, sx0, sga1, sgb1, sf1,
                wait_f)

    i_start(0, sia0, sib0, sx0)
    sec0(0, False)
    sec1(1, False)

    @pl.loop(2, SEC, step=2)
    def _(S):
        sec0(S, True)
        sec1(S + 1, True)

    for src_r, dst_r, sem in f_descs(SEC - 2, 0, sf0):
        pltpu.make_async_copy(src_r, dst_r, sem).wait()
    for src_r, dst_r, sem in f_descs(SEC - 1, 1, sf1):
        pltpu.make_async_copy(src_r, dst_r, sem).wait()
    i_wait(SEC, sia0, sib0, sx0)


def _split_cols(t):
    return jnp.concatenate([t[:, :128], t[:, 128:]], axis=0)


def kernel(x, edge_index, edge_attr, params):
    src = edge_index[0]
    dst = edge_index[1]
    src2 = jnp.concatenate([src, src + N])
    dst2 = jnp.concatenate([dst, dst + N])
    zrows = jnp.zeros((RPT, 128), jnp.float32)

    def _as_i32(t):
        return jax.lax.bitcast_convert_type(
            t.reshape(t.shape[0], 128, 2), jnp.int32)

    def _as_bf16(t):
        return jax.lax.bitcast_convert_type(t, jnp.bfloat16).reshape(-1, H)
    p0, p1, p2 = params['layers']
    we_s0, we_d0, we_e0 = jnp.split(p0['W_e'], 3, axis=0)
    we_s1, we_d1, we_e1 = jnp.split(p1['W_e'], 3, axis=0)
    wcm = params['W_ee'] @ p0['W_msg']
    wce = params['W_ee'] @ we_e0
    bcm = (params['b_ee'] @ p0['W_msg']).reshape(1, H)
    bce = (params['b_ee'] @ we_e0).reshape(1, H)
    w_pred_pad = jnp.pad(params['W_pred'], ((0, 0), (0, 128 - OUT)))

    h0, hm0 = _prep(x, params['W_ne'], params['b_ne'].reshape(1, H),
                    p0['W_msg'])
    em0, ee0 = _edge_mm0(edge_attr, wcm, wce, bcm, bce)
    agg0 = _sc_scatter(_split_cols(hm0), em0, src2, dst, zrows)
    h1, hm1, hs0, hd0 = _node_mid(h0, agg0, p0['W_self'],
                                  p0['b_h'].reshape(1, H),
                                  p1['W_msg'], we_s0, we_d0)
    gs0, gd0 = _sc_gather(_as_i32(hs0), _as_i32(hd0), src, dst)
    em1, ee1 = _edge_mm(ee0, _as_bf16(gs0), _as_bf16(gd0),
                        p0['b_e'].reshape(1, H),
                        p1['W_msg'], we_e1, want_ee=True)
    agg1 = _sc_scatter(_split_cols(hm1), em1, src2, dst, zrows)
    h2, hm2, hs1, hd1 = _node_mid(h1, agg1, p1['W_self'],
                                  p1['b_h'].reshape(1, H),
                                  p2['W_msg'], we_s1, we_d1)
    gs1, gd1 = _sc_gather(_as_i32(hs1), _as_i32(hd1), src, dst)
    em2, _ = _edge_mm(ee1, _as_bf16(gs1), _as_bf16(gd1),
                      p1['b_e'].reshape(1, H),
                      p2['W_msg'], None, want_ee=False)
    agg2 = _sc_scatter(_split_cols(hm2), em2, src2, dst, zrows)
    out = _node_last(h2, agg2, p2['W_self'], p2['b_h'].reshape(1, H),
                     w_pred_pad)
    return out[:, :OUT] + params['b_pred']


# speedup vs baseline: 2.6989x; 2.6989x over previous
import functools

import jax
import jax.numpy as jnp
from jax import lax
from jax.experimental import pallas as pl
from jax.experimental.pallas import tpu as pltpu
from jax.experimental.pallas import tpu_sc as plsc

N = 10000
E = 320000
H = 256
OUT = 1
BE = 2560

NS = 16
NFT = 10
RPT = N // NFT
BN = 2000

KS = 80
SCH = (E // NS) // KS

_sc_mesh = plsc.VectorSubcoreMesh(core_axis_name="c", subcore_axis_name="s")


def _edge_mm0_kernel(ea_ref, wm_ref, we_ref, bm_ref, be_ref, em_ref, ee_ref):
    ea = ea_ref[...]
    em_ref[...] = jnp.dot(ea, wm_ref[...],
                          preferred_element_type=jnp.float32) + bm_ref[...]
    ee_ref[...] = (jnp.dot(ea, we_ref[...], preferred_element_type=jnp.float32)
                   + be_ref[...]).astype(jnp.bfloat16)


def _edge_mm0(ea, wcm, wce, bcm, bce):
    de = ea.shape[1]
    return pl.pallas_call(
        _edge_mm0_kernel,
        grid=(E // BE,),
        in_specs=[
            pl.BlockSpec((BE, de), lambda i: (i, 0)),
            pl.BlockSpec((de, H), lambda i: (0, 0)),
            pl.BlockSpec((de, H), lambda i: (0, 0)),
            pl.BlockSpec((1, H), lambda i: (0, 0)),
            pl.BlockSpec((1, H), lambda i: (0, 0)),
        ],
        out_specs=[
            pl.BlockSpec((BE, H), lambda i: (i, 0)),
            pl.BlockSpec((BE, H), lambda i: (i, 0)),
        ],
        out_shape=[
            jax.ShapeDtypeStruct((E, H), jnp.float32),
            jax.ShapeDtypeStruct((E, H), jnp.bfloat16),
        ],
    )(ea, wcm, wce, bcm, bce)


def _unpack_tau(x_i32):
    lo = jax.lax.bitcast_convert_type(
        jax.lax.shift_left(x_i32, 16), jnp.float32)
    hi = jax.lax.bitcast_convert_type(
        jax.lax.bitwise_and(x_i32, jnp.int32(-65536)), jnp.float32)
    return jnp.concatenate([lo, hi], axis=1)


def _edge_mm_kernel(ee_ref, gs_ref, gd_ref, be_ref, wm_ref, we_ref,
                    em_ref, eeo_ref):
    e = jax.nn.relu(
        ee_ref[...].astype(jnp.float32)
        + _unpack_tau(gs_ref[...])
        + _unpack_tau(gd_ref[...])
        + be_ref[...])
    em_ref[...] = jnp.dot(e, wm_ref[...], preferred_element_type=jnp.float32)
    if eeo_ref is not None:
        eeo_ref[...] = jnp.dot(
            e, we_ref[...], preferred_element_type=jnp.float32
        ).astype(jnp.bfloat16)


def _edge_mm(ee_prev, gs, gd, b_e, w_msg, w_ee, want_ee):
    nblk = E // BE
    out_specs = [pl.BlockSpec((BE, H), lambda i: (i, 0))]
    out_shape = [jax.ShapeDtypeStruct((E, H), jnp.float32)]
    in_specs = [
        pl.BlockSpec((BE, H), lambda i: (i, 0)),
        pl.BlockSpec((BE, 128), lambda i: (i, 0)),
        pl.BlockSpec((BE, 128), lambda i: (i, 0)),
        pl.BlockSpec((1, H), lambda i: (0, 0)),
        pl.BlockSpec((H, H), lambda i: (0, 0)),
    ]
    args = [ee_prev, gs, gd, b_e, w_msg]
    if want_ee:
        out_specs.append(pl.BlockSpec((BE, H), lambda i: (i, 0)))
        out_shape.append(jax.ShapeDtypeStruct((E, H), jnp.bfloat16))
        in_specs.append(pl.BlockSpec((H, H), lambda i: (0, 0)))
        args.append(w_ee)
        body = _edge_mm_kernel
    else:
        def body(ee_ref, gs_ref, gd_ref, be_ref, wm_ref, em_ref):
            _edge_mm_kernel(ee_ref, gs_ref, gd_ref, be_ref, wm_ref, None,
                            em_ref, None)
    res = pl.pallas_call(
        body,
        grid=(nblk,),
        in_specs=in_specs,
        out_specs=out_specs,
        out_shape=out_shape,
    )(*args)
    return res if want_ee else (res[0], None)


def _prep_kernel(x_ref, wn_ref, bn_ref, wm_ref, h_ref, hm_ref):
    h = jnp.dot(x_ref[...], wn_ref[...],
                preferred_element_type=jnp.float32) + bn_ref[...]
    h_ref[...] = h
    hm_ref[...] = jnp.dot(h, wm_ref[...], preferred_element_type=jnp.float32)


def _prep(x, w_ne, b_ne, w_msg):
    df = x.shape[1]
    return pl.pallas_call(
        _prep_kernel,
        grid=(N // BN,),
        in_specs=[
            pl.BlockSpec((BN, df), lambda i: (i, 0)),
            pl.BlockSpec((df, H), lambda i: (0, 0)),
            pl.BlockSpec((1, H), lambda i: (0, 0)),
            pl.BlockSpec((H, H), lambda i: (0, 0)),
        ],
        out_specs=[
            pl.BlockSpec((BN, H), lambda i: (i, 0)),
            pl.BlockSpec((BN, H), lambda i: (i, 0)),
        ],
        out_shape=[
            jax.ShapeDtypeStruct((N, H), jnp.float32),
            jax.ShapeDtypeStruct((N, H), jnp.float32),
        ],
    )(x, w_ne, b_ne, w_msg)


def _node_mid_kernel(h_ref, aga_ref, agb_ref, ws_ref, bh_ref, wm_ref,
                     wes_ref, wed_ref, hn_ref, hm_ref, hs_ref, hd_ref):
    hn = jax.nn.relu(
        jnp.dot(h_ref[...], ws_ref[...], preferred_element_type=jnp.float32)
        + jnp.concatenate([aga_ref[...], agb_ref[...]], axis=1)
        + bh_ref[...])
    hn_ref[...] = hn
    if hm_ref is not None:
        hm_ref[...] = jnp.dot(hn, wm_ref[...],
                              preferred_element_type=jnp.float32)
    hs_ref[...] = jnp.dot(
        hn, wes_ref[...], preferred_element_type=jnp.float32
    ).astype(jnp.bfloat16)
    hd_ref[...] = jnp.dot(
        hn, wed_ref[...], preferred_element_type=jnp.float32
    ).astype(jnp.bfloat16)


def _node_mid(h, agg_flat, w_self, b_h, w_msg_next, we_s, we_d):
    nblk = N // BN
    return pl.pallas_call(
        _node_mid_kernel,
        grid=(nblk,),
        in_specs=[
            pl.BlockSpec((BN, H), lambda i: (i, 0)),
            pl.BlockSpec((BN, 128), lambda i: (i, 0)),
            pl.BlockSpec((BN, 128), lambda i: (i + nblk, 0)),
            pl.BlockSpec((H, H), lambda i: (0, 0)),
            pl.BlockSpec((1, H), lambda i: (0, 0)),
            pl.BlockSpec((H, H), lambda i: (0, 0)),
            pl.BlockSpec((H, H), lambda i: (0, 0)),
            pl.BlockSpec((H, H), lambda i: (0, 0)),
        ],
        out_specs=[pl.BlockSpec((BN, H), lambda i: (i, 0))] * 4,
        out_shape=[jax.ShapeDtypeStruct((N, H), jnp.float32)] * 2
        + [jax.ShapeDtypeStruct((N, H), jnp.bfloat16)] * 2,
    )(h, agg_flat, agg_flat, w_self, b_h, w_msg_next, we_s, we_d)


def _node_last_kernel(h_ref, aga_ref, agb_ref, ws_ref, bh_ref, wp_ref,
                      o_ref):
    hn = jax.nn.relu(
        jnp.dot(h_ref[...], ws_ref[...], preferred_element_type=jnp.float32)
        + jnp.concatenate([aga_ref[...], agb_ref[...]], axis=1)
        + bh_ref[...])
    o_ref[...] = jnp.dot(hn, wp_ref[...], preferred_element_type=jnp.float32)


def _node_last(h, agg_flat, w_self, b_h, w_pred_pad):
    nblk = N // BN
    return pl.pallas_call(
        _node_last_kernel,
        grid=(nblk,),
        in_specs=[
            pl.BlockSpec((BN, H), lambda i: (i, 0)),
            pl.BlockSpec((BN, 128), lambda i: (i, 0)),
            pl.BlockSpec((BN, 128), lambda i: (i + nblk, 0)),
            pl.BlockSpec((H, H), lambda i: (0, 0)),
            pl.BlockSpec((1, H), lambda i: (0, 0)),
            pl.BlockSpec((H, 128), lambda i: (0, 0)),
        ],
        out_specs=[pl.BlockSpec((BN, 128), lambda i: (i, 0))],
        out_shape=[jax.ShapeDtypeStruct((N, 128), jnp.float32)],
    )(h, agg_flat, agg_flat, w_self, b_h, w_pred_pad)[0]


def _off(v, m):
    return pl.multiple_of(v, m)


@functools.partial(
    pl.kernel,
    out_type=jax.ShapeDtypeStruct((2 * N, 128), jnp.float32),
    mesh=_sc_mesh,
    scratch_types=[
        pltpu.VMEM((KS,), jnp.int32),
        pltpu.VMEM((KS,), jnp.int32),
        pltpu.VMEM((KS,), jnp.int32),
        pltpu.VMEM((KS,), jnp.int32),
        pltpu.VMEM((KS, 128), jnp.float32),
        pltpu.VMEM((KS, 128), jnp.float32),
        pltpu.VMEM((KS, 128), jnp.float32),
        pltpu.VMEM((KS, 128), jnp.float32),
        pltpu.VMEM_SHARED((N, 128), jnp.float32),
    ] + [pltpu.SemaphoreType.DMA] * 8,
)
def _sc_scatter(hm_hbm, em_hbm, src2_hbm, dst_hbm, z_hbm, out_hbm,
                si0, si1, di0, di1, ga0, ga1, em0, em1, acc,
                ssi0, ssi1, sdi0, sdi1, sg0, sg1, se0, se1):
    c = lax.axis_index("c")
    s = lax.axis_index("s")

    @pl.when(s < NFT)
    def _():
        pltpu.sync_copy(z_hbm, acc.at[pl.ds(s * RPT, RPT)])

    tbase = (c * NS + s) * SCH
    dbase = s * SCH

    def sioff(j):
        return _off(jnp.minimum(tbase + j, tbase + SCH - 1) * KS, 8)

    def dioff(j):
        return _off(jnp.minimum(dbase + j, dbase + SCH - 1) * KS, 8)

    def si_start(j, si, sem):
        pltpu.async_copy(src2_hbm.at[pl.ds(sioff(j), KS)], si, sem)

    def si_wait(j, si, sem):
        pltpu.make_async_copy(src2_hbm.at[pl.ds(sioff(j), KS)], si, sem).wait()

    def di_start(j, di, sem):
        pltpu.async_copy(dst_hbm.at[pl.ds(dioff(j), KS)], di, sem)

    def di_wait(j, di, sem):
        pltpu.make_async_copy(dst_hbm.at[pl.ds(dioff(j), KS)], di, sem).wait()

    def g_start(j, si, ga, em, sg, se):
        pltpu.async_copy(hm_hbm.at[si], ga, sg)
        pltpu.async_copy(
            em_hbm.at[pl.ds(dioff(j), KS), pl.ds(_off(c * 128, 128), 128)],
            em, se)

    def g_wait(j, si, ga, em, sg, se):
        pltpu.make_async_copy(hm_hbm.at[si], ga, sg).wait()
        pltpu.make_async_copy(
            em_hbm.at[pl.ds(dioff(j), KS), pl.ds(_off(c * 128, 128), 128)],
            em, se).wait()

    si_start(0, si0, ssi0)
    si_start(1, si1, ssi1)
    di_start(0, di0, sdi0)
    di_start(1, di1, sdi1)
    si_wait(0, si0, ssi0)
    g_start(0, si0, ga0, em0, sg0, se0)
    si_wait(1, si1, ssi1)
    g_start(1, si1, ga1, em1, sg1, se1)
    plsc.subcore_barrier()

    def chunk(j, si, di, ga, em, ssi, sdi, sg, se, jn):
        g_wait(j, si, ga, em, sg, se)
        si_start(jn, si, ssi)
        di_wait(j, di, sdi)
        pltpu.sync_copy(ga, acc.at[di], add=True)
        pltpu.sync_copy(em, acc.at[di], add=True)
        di_start(jn, di, sdi)
        si_wait(jn, si, ssi)
        g_start(jn, si, ga, em, sg, se)

    @pl.loop(0, SCH, step=2)
    def _(j):
        chunk(j, si0, di0, ga0, em0, ssi0, sdi0, sg0, se0, j + 2)
        chunk(j + 1, si1, di1, ga1, em1, ssi1, sdi1, sg1, se1, j + 3)

    g_wait(SCH, si0, ga0, em0, sg0, se0)
    g_wait(SCH + 1, si1, ga1, em1, sg1, se1)
    di_wait(SCH, di0, sdi0)
    di_wait(SCH + 1, di1, sdi1)

    plsc.subcore_barrier()

    @pl.when(s < NFT)
    def _():
        pltpu.sync_copy(acc.at[pl.ds(s * RPT, RPT)],
                        out_hbm.at[pl.ds(c * N + s * RPT, RPT)])


KG = 40
SUP = 5
SROWS = SUP * KG
EPTG = E // (2 * NS)
SEC = EPTG // SROWS


@functools.partial(
    pl.kernel,
    out_type=[jax.ShapeDtypeStruct((E, 128), jnp.int32),
              jax.ShapeDtypeStruct((E, 128), jnp.int32)],
    mesh=_sc_mesh,
    scratch_types=[
        pltpu.VMEM((SROWS,), jnp.int32),
        pltpu.VMEM((SROWS,), jnp.int32),
        pltpu.VMEM((SROWS,), jnp.int32),
        pltpu.VMEM((SROWS,), jnp.int32),
        pltpu.VMEM((2 * SROWS, 128), jnp.int32),
        pltpu.VMEM((2 * SROWS, 128), jnp.int32),
    ] + [pltpu.SemaphoreType.DMA] * 8,
)
def _sc_gather(hs_hbm, hd_hbm, src_hbm, dst_hbm, gs_hbm, gd_hbm,
               sia0, sia1, sib0, sib1, ra, rb,
               sx0, sx1, sga0, sga1, sgb0, sgb1, sf0, sf1):
    c = lax.axis_index("c")
    s = lax.axis_index("s")
    ebase = (c * NS + s) * EPTG

    def eoff(S):
        return _off(ebase + jnp.minimum(S, SEC - 1) * SROWS, 8)

    def i_start(S, sia, sib, sx):
        pltpu.async_copy(src_hbm.at[pl.ds(eoff(S), SROWS)], sia, sx)
        pltpu.async_copy(dst_hbm.at[pl.ds(eoff(S), SROWS)], sib, sx)

    def i_wait(S, sia, sib, sx):
        pltpu.make_async_copy(
            src_hbm.at[pl.ds(eoff(S), SROWS)], sia, sx).wait()
        pltpu.make_async_copy(
            dst_hbm.at[pl.ds(eoff(S), SROWS)], sib, sx).wait()

    def g_descs(P, sia, sib, sga, sgb):
        for kk in range(SUP):
            yield (hs_hbm.at[sia.at[pl.ds(kk * KG, KG)]],
                   ra.at[pl.ds(P * SROWS + kk * KG, KG)], sga)
            yield (hd_hbm.at[sib.at[pl.ds(kk * KG, KG)]],
                   rb.at[pl.ds(P * SROWS + kk * KG, KG)], sgb)

    def f_descs(S, P, sf):
        yield (ra.at[pl.ds(P * SROWS, SROWS)],
               gs_hbm.at[pl.ds(eoff(S), SROWS)], sf)
        yield (rb.at[pl.ds(P * SROWS, SROWS)],
               gd_hbm.at[pl.ds(eoff(S), SROWS)], sf)

    def section(S, P, sia, sib, sx, sia_n, sib_n, sx_n, sga, sgb, sf,
                wait_f):
        if wait_f:
            for src_r, dst_r, sem in f_descs(S - 2, P, sf):
                pltpu.make_async_copy(src_r, dst_r, sem).wait()
        i_wait(S, sia, sib, sx)
        for src_r, dst_r, sem in g_descs(P, sia, sib, sga, sgb):
            pltpu.async_copy(src_r, dst_r, sem)
        i_start(S + 1, sia_n, sib_n, sx_n)
        for src_r, dst_r, sem in g_descs(P, sia, sib, sga, sgb):
            pltpu.make_async_copy(src_r, dst_r, sem).wait()
        for src_r, dst_r, sem in f_descs(S, P, sf):
            pltpu.async_copy(src_r, dst_r, sem)

    def sec0(S, wait_f):
        section(S, 0, sia0, sib0, sx0, sia1, sib1, sx1, sga0, sgb0, sf0,
                wait_f)

    def sec1(S, wait_f):
        section(S, 1, sia1, sib1, sx1, sia0, sib0, sx0, sga1, sgb1, sf1,
                wait_f)

    i_start(0, sia0, sib0, sx0)
    sec0(0, False)
    sec1(1, False)

    @pl.loop(2, SEC, step=2)
    def _(S):
        sec0(S, True)
        sec1(S + 1, True)

    for src_r, dst_r, sem in f_descs(SEC - 2, 0, sf0):
        pltpu.make_async_copy(src_r, dst_r, sem).wait()
    for src_r, dst_r, sem in f_descs(SEC - 1, 1, sf1):
        pltpu.make_async_copy(src_r, dst_r, sem).wait()
    i_wait(SEC, sia0, sib0, sx0)


def _split_cols(t):
    return jnp.concatenate([t[:, :128], t[:, 128:]], axis=0)


def kernel(x, edge_index, edge_attr, params):
    src = edge_index[0]
    dst = edge_index[1]
    src2 = jnp.concatenate([src, src + N])
    dst2 = jnp.concatenate([dst, dst + N])
    zrows = jnp.zeros((RPT, 128), jnp.float32)

    def _as_i32(t):
        return jax.lax.bitcast_convert_type(
            t.reshape(t.shape[0], 128, 2), jnp.int32)

    tau = jnp.concatenate([jnp.arange(0, H, 2), jnp.arange(1, H, 2)])
    p0, p1, p2 = params['layers']
    we_s0, we_d0, we_e0 = jnp.split(p0['W_e'], 3, axis=0)
    we_s1, we_d1, we_e1 = jnp.split(p1['W_e'], 3, axis=0)
    wcm = params['W_ee'] @ p0['W_msg']
    wce = (params['W_ee'] @ we_e0)[:, tau]
    bcm = (params['b_ee'] @ p0['W_msg']).reshape(1, H)
    bce = (params['b_ee'] @ we_e0)[tau].reshape(1, H)
    w_pred_pad = jnp.pad(params['W_pred'], ((0, 0), (0, 128 - OUT)))

    h0, hm0 = _prep(x, params['W_ne'], params['b_ne'].reshape(1, H),
                    p0['W_msg'])
    em0, ee0 = _edge_mm0(edge_attr, wcm, wce, bcm, bce)
    agg0 = _sc_scatter(_split_cols(hm0), em0, src2, dst, zrows)
    h1, hm1, hs0, hd0 = _node_mid(h0, agg0, p0['W_self'],
                                  p0['b_h'].reshape(1, H),
                                  p1['W_msg'], we_s0, we_d0)
    gs0, gd0 = _sc_gather(_as_i32(hs0), _as_i32(hd0), src, dst)
    em1, ee1 = _edge_mm(ee0, gs0, gd0,
                        p0['b_e'][tau].reshape(1, H),
                        p1['W_msg'][tau, :], we_e1[tau][:, tau],
                        want_ee=True)
    agg1 = _sc_scatter(_split_cols(hm1), em1, src2, dst, zrows)
    h2, hm2, hs1, hd1 = _node_mid(h1, agg1, p1['W_self'],
                                  p1['b_h'].reshape(1, H),
                                  p2['W_msg'], we_s1, we_d1)
    gs1, gd1 = _sc_gather(_as_i32(hs1), _as_i32(hd1), src, dst)
    em2, _ = _edge_mm(ee1, gs1, gd1,
                      p1['b_e'][tau].reshape(1, H),
                      p2['W_msg'][tau, :], None, want_ee=False)
    agg2 = _sc_scatter(_split_cols(hm2), em2, src2, dst, zrows)
    out = _node_last(h2, agg2, p2['W_self'], p2['b_h'].reshape(1, H),
                     w_pred_pad)
    return out[:, :OUT] + params['b_pred']
